# baseline (speedup 1.0000x reference)
import functools

import jax
import jax.numpy as jnp
from jax import lax
from jax.experimental import pallas as pl
from jax.experimental.pallas import tpu as pltpu
from jax.experimental.pallas import tpu_sc as plsc

_NUM_CORES = 2
_NUM_SUBCORES = 16
_NW = _NUM_CORES * _NUM_SUBCORES

_BT = 128
_NBUF = 4
_NIDX = 8


def _embedding_lookup(tokT, table, N0, N1, D):
    nb0 = N0 // _BT
    n_blocks = nb0 * N1
    nw = n_blocks // _NW
    tiles_f = D // 8
    tile_sz = 8 * _BT
    pos_sz = tiles_f * nb0 * tile_sz

    mesh = plsc.VectorSubcoreMesh(
        core_axis_name="c",
        subcore_axis_name="s",
        num_cores=_NUM_CORES,
        num_subcores=_NUM_SUBCORES,
    )

    @functools.partial(
        pl.kernel,
        out_type=jax.ShapeDtypeStruct((N1 * pos_sz,), jnp.float32),
        mesh=mesh,
        scratch_types=[
            pltpu.VMEM((_NIDX, _BT), jnp.int32),
            pltpu.VMEM((_NBUF * _BT, D), jnp.float32),
            [pltpu.VMEM((tiles_f * tile_sz,), jnp.float32)] * _NBUF,
            [pltpu.SemaphoreType.DMA] * _NIDX,
            [pltpu.SemaphoreType.DMA] * _NBUF,
            [pltpu.SemaphoreType.DMA] * _NBUF,
        ],
        compiler_params=pltpu.CompilerParams(use_tc_tiling_on_sc=False,
                                             needs_layout_passes=False),
    )
    def k(tok_hbm, table_hbm, out_hbm, idx_v, rows_v, t_v,
          sem_i, sem_g, sem_o):
        wid = lax.axis_index("s") * _NUM_CORES + lax.axis_index("c")
        g0 = wid * nw

        lane = lax.iota(jnp.int32, 16)
        cbase0 = (lane // 8) * tile_sz + (lane % 8) * _BT
        cbase1 = cbase0 + 2 * tile_sz

        def idx_copy(c, ib):
            g = g0 + c
            b1, tb = g // nb0, g % nb0
            return pltpu.make_async_copy(
                tok_hbm.at[b1, pl.ds(tb * _BT, _BT)], idx_v.at[ib],
                sem_i[ib])

        def gather(c, ib, b):
            return pltpu.make_async_copy(
                table_hbm.at[idx_v.at[ib]],
                rows_v.at[pl.ds(b * _BT, _BT), :], sem_g[b])

        def out_copies(c, b):
            g = g0 + c
            b1, tb = g // nb0, g % nb0
            base = b1 * pos_sz + tb * tile_sz
            return [
                pltpu.make_async_copy(
                    t_v[b].at[pl.ds(tc * tile_sz, tile_sz)],
                    out_hbm.at[pl.ds(base + tc * (nb0 * tile_sz), tile_sz)],
                    sem_o[b])
                for tc in range(tiles_f)
            ]

        def transpose(b):
            dst = t_v[b]
            rbase = b * _BT

            @plsc.parallel_loop(0, D * (_BT // 16), 1, unroll=16)
            def _(i):
                c = i // 8
                biog = i % 8
                rows = lax.broadcast(rbase + biog * 16, (16,)) + lane
                cols = lax.broadcast(c, (16,))
                y = plsc.load_gather(rows_v, [rows, cols])
                off = (c // 8) * tile_sz + (c % 8) * _BT + biog * 16
                dst[pl.ds(off, 16)] = y

        def step(c, p, fire_i, wait_i, wait_o):
            if fire_i:
                idx_copy(c + 3, (p + 3) % _NIDX).start()
            if wait_i:
                idx_copy(c + 2, (p + 2) % _NIDX).wait()
            if wait_o:
                for oc in out_copies(c - 2, (p + 2) % _NBUF):
                    oc.wait()
            if wait_i:
                gather(c + 2, (p + 2) % _NIDX, (p + 2) % _NBUF).start()
            gather(c, p % _NIDX, p % _NBUF).wait()
            transpose(p % _NBUF)
            for oc in out_copies(c, p % _NBUF):
                oc.start()

        for c in (0, 1, 2):
            idx_copy(c, c).start()
        for c in (0, 1):
            idx_copy(c, c).wait()
            gather(c, c, c).start()
        step(0, 0, True, True, False)
        step(1, 1, True, True, False)

        @pl.loop(2, nw - 6, step=8)
        def steady(cv):
            for b_off in range(8):
                step(cv + b_off, 2 + b_off, True, True, True)

        for c in range(nw - 6, nw):
            step(c, c, c + 3 < nw, c + 2 < nw, True)
        for c in (nw - 2, nw - 1):
            for oc in out_copies(c, c % _NBUF):
                oc.wait()

    return k(tokT, table)


def kernel(token_ids, weights):
    N0, N1 = token_ids.shape
    V, D = weights.shape
    tokT = token_ids.T
    w_flat = lax.optimization_barrier(weights.reshape(V * D))
    outf = _embedding_lookup(tokT, w_flat.reshape(V, D), N0, N1, D)
    out5 = outf.reshape(N1, D // 8, N0 // _BT, 8, _BT)
    t = out5.transpose((2, 4, 0, 1, 3))
    return t.reshape(N0, N1, D)


# speedup vs baseline: 1.1073x; 1.1073x over previous
import functools

import jax
import jax.numpy as jnp
from jax import lax
from jax.experimental import pallas as pl
from jax.experimental.pallas import tpu as pltpu
from jax.experimental.pallas import tpu_sc as plsc

_NUM_CORES = 2
_NUM_SUBCORES = 16
_NW = _NUM_CORES * _NUM_SUBCORES

_BT = 128
_NBUF = 4
_NIDX = 8


def _embedding_lookup(tokT, table, N0, N1, D):
    nb0 = N0 // _BT
    n_blocks = nb0 * N1
    nw = n_blocks // _NW
    tiles_f = D // 8
    tile_sz = 8 * _BT
    pos_sz = tiles_f * nb0 * tile_sz

    mesh = plsc.VectorSubcoreMesh(
        core_axis_name="c",
        subcore_axis_name="s",
        num_cores=_NUM_CORES,
        num_subcores=_NUM_SUBCORES,
    )

    @functools.partial(
        pl.kernel,
        out_type=jax.ShapeDtypeStruct((N1 * pos_sz,), jnp.float32),
        mesh=mesh,
        scratch_types=[
            pltpu.VMEM((_NIDX, _BT), jnp.int32),
            pltpu.VMEM((_NBUF * _BT, D), jnp.float32),
            [pltpu.VMEM((tiles_f * tile_sz,), jnp.float32)] * _NBUF,
            [pltpu.SemaphoreType.DMA] * _NIDX,
            [pltpu.SemaphoreType.DMA] * _NBUF,
            [pltpu.SemaphoreType.DMA] * _NBUF,
        ],
        compiler_params=pltpu.CompilerParams(use_tc_tiling_on_sc=False,
                                             needs_layout_passes=False),
    )
    def k(tok_hbm, table_hbm, out_hbm, idx_v, rows_v, t_v,
          sem_i, sem_g, sem_o):
        wid = lax.axis_index("s") * _NUM_CORES + lax.axis_index("c")
        g0 = wid * nw

        lane = lax.iota(jnp.int32, 16)
        cbase0 = (lane // 8) * tile_sz + (lane % 8) * _BT
        cbase1 = cbase0 + 2 * tile_sz

        def idx_copy(c, ib):
            g = g0 + c
            b1, tb = g // nb0, g % nb0
            return pltpu.make_async_copy(
                tok_hbm.at[b1, pl.ds(tb * _BT, _BT)], idx_v.at[ib],
                sem_i[ib])

        def gather(c, ib, b):
            return pltpu.make_async_copy(
                table_hbm.at[idx_v.at[ib]],
                rows_v.at[pl.ds(b * _BT, _BT), :], sem_g[b])

        def out_copies(c, b):
            g = g0 + c
            b1, tb = g // nb0, g % nb0
            base = b1 * pos_sz + tb * tile_sz
            return [
                pltpu.make_async_copy(
                    t_v[b].at[pl.ds(tc * tile_sz, tile_sz)],
                    out_hbm.at[pl.ds(base + tc * (nb0 * tile_sz), tile_sz)],
                    sem_o[b])
                for tc in range(tiles_f)
            ]

        def transpose(b):
            dst = t_v[b]
            rbase = b * _BT

            @plsc.parallel_loop(0, _BT // 16, 1)
            def _(biog):
                boff = biog * 16
                rows = lax.broadcast(rbase, (16,)) + boff + lane
                for c in range(D):
                    cols = jnp.full((16,), c, jnp.int32)
                    y = plsc.load_gather(rows_v, [rows, cols])
                    off = (c // 8) * tile_sz + (c % 8) * _BT
                    dst[pl.ds(off + boff, 16)] = y

        def step(c, p, fire_i, wait_i, wait_o):
            if fire_i:
                idx_copy(c + 3, (p + 3) % _NIDX).start()
            if wait_i:
                idx_copy(c + 2, (p + 2) % _NIDX).wait()
            if wait_o:
                for oc in out_copies(c - 2, (p + 2) % _NBUF):
                    oc.wait()
            if wait_i:
                gather(c + 2, (p + 2) % _NIDX, (p + 2) % _NBUF).start()
            gather(c, p % _NIDX, p % _NBUF).wait()
            transpose(p % _NBUF)
            for oc in out_copies(c, p % _NBUF):
                oc.start()

        for c in (0, 1, 2):
            idx_copy(c, c).start()
        for c in (0, 1):
            idx_copy(c, c).wait()
            gather(c, c, c).start()
        step(0, 0, True, True, False)
        step(1, 1, True, True, False)

        @pl.loop(2, nw - 6, step=8)
        def steady(cv):
            for b_off in range(8):
                step(cv + b_off, 2 + b_off, True, True, True)

        for c in range(nw - 6, nw):
            step(c, c, c + 3 < nw, c + 2 < nw, True)
        for c in (nw - 2, nw - 1):
            for oc in out_copies(c, c % _NBUF):
                oc.wait()

    return k(tokT, table)


def kernel(token_ids, weights):
    N0, N1 = token_ids.shape
    V, D = weights.shape
    tokT = token_ids.T
    w_flat = lax.optimization_barrier(weights.reshape(V * D))
    outf = _embedding_lookup(tokT, w_flat.reshape(V, D), N0, N1, D)
    out5 = outf.reshape(N1, D // 8, N0 // _BT, 8, _BT)
    t = out5.transpose((2, 4, 0, 1, 3))
    return t.reshape(N0, N1, D)
